# baseline (speedup 1.0000x reference)
import functools

import jax
import jax.numpy as jnp
from jax import lax
from jax.experimental import pallas as pl
from jax.experimental.pallas import tpu as pltpu
from jax.experimental.pallas import tpu_sc as plsc

N_CORES = 2
N_SUB = 16
LANES = 128
BW = 128
NSLOT = 3


def _mesh():
    return plsc.VectorSubcoreMesh(
        core_axis_name="c", subcore_axis_name="s",
        num_cores=N_CORES, num_subcores=N_SUB)


def _fused_body(n, nb, passes, table_hbm, gidx_hbm, sidx_hbm, zeros_hbm,
                out_hbm, ring_v, slots_v,
                d0, d1, d2, s0, s1, s2, g0, g1, g2, x0, x1, x2, acc):
    dsem = (d0, d1, d2)
    ssem = (s0, s1, s2)
    gsem = (g0, g1, g2)
    xsem = (x0, x1, x2)
    c = lax.axis_index("c")
    s = lax.axis_index("s")
    wb = (n // N_SUB) & ~7
    wb_last = n - (N_SUB - 1) * wb

    def dwait(sem, j):
        pltpu.make_async_copy(table_hbm.at[pl.ds(0, BW)], slots_v.at[j],
                              sem[j]).wait()

    def iwait(sem, j):
        pltpu.make_async_copy(gidx_hbm.at[pl.ds(0, 1)], ring_v.at[pl.ds(j, 1)],
                              sem[j]).wait()

    if any(kind == "ones" for kind, _, _, _ in passes):
        def fill_ones(k, _):
            slots_v[0, k // 8, pl.ds((k % 8) * 16, 16)] = jnp.ones(
                (16,), jnp.float32)
            return 0
        lax.fori_loop(0, 128 * 8, fill_ones, 0)

    for kind, gv, sv, ob in passes:
        if kind == "ones":
            srow = (c * N_SUB + s) * nb
        else:
            srow = (sv * N_SUB + s) * nb
            grow = ((gv + c) * N_SUB + s) * nb

        @pl.when(s < N_SUB - 1)
        def _():
            pltpu.sync_copy(zeros_hbm.at[pl.ds(0, wb)], acc.at[pl.ds(s * wb, wb)])

        @pl.when(s == N_SUB - 1)
        def _():
            pltpu.sync_copy(zeros_hbm.at[pl.ds(0, wb_last)],
                            acc.at[pl.ds((N_SUB - 1) * wb, wb_last)])
        plsc.subcore_barrier()

        if kind == "ones":
            for j in range(2):
                pltpu.async_copy(sidx_hbm.at[pl.ds(srow + j, 1)],
                                 ring_v.at[pl.ds(4 + j, 1)], xsem[j])

            def ostep(b, j):
                j2 = (j + 2) % NSLOT

                @pl.when(b + 2 < nb)
                def _():
                    pltpu.async_copy(sidx_hbm.at[pl.ds(srow + b + 2, 1)],
                                     ring_v.at[pl.ds(4 + j2, 1)], xsem[j2])
                iwait(xsem, j)
                pltpu.sync_copy(slots_v.at[0], acc.at[ring_v.at[4 + j]],
                                add=True)

            def ogroup(i, _):
                for j in range(NSLOT):
                    ostep(i * NSLOT + j, j)
                return 0
            lax.fori_loop(0, nb // NSLOT, ogroup, 0)
            for j in range(nb - nb // NSLOT * NSLOT):
                ostep(nb // NSLOT * NSLOT + j, j)
        else:
            for j in range(3):
                pltpu.async_copy(gidx_hbm.at[pl.ds(grow + j, 1)],
                                 ring_v.at[pl.ds(j, 1)], gsem[j])
            for j in range(2):
                pltpu.async_copy(sidx_hbm.at[pl.ds(srow + j, 1)],
                                 ring_v.at[pl.ds(4 + j, 1)], xsem[j])
            for j in range(2):
                iwait(gsem, j)
                pltpu.async_copy(table_hbm.at[ring_v.at[j]], slots_v.at[j],
                                 dsem[j])

            def step(b, j):
                j2 = (j + 2) % NSLOT

                @pl.when(b >= 1)
                def _():
                    dwait(ssem, j2)

                @pl.when(b + 2 < nb)
                def _():
                    pltpu.async_copy(sidx_hbm.at[pl.ds(srow + b + 2, 1)],
                                     ring_v.at[pl.ds(4 + j2, 1)], xsem[j2])
                    iwait(gsem, j2)
                    pltpu.async_copy(table_hbm.at[ring_v.at[j2]],
                                     slots_v.at[j2], dsem[j2])
                dwait(dsem, j)

                @pl.when(b + 3 < nb)
                def _():
                    pltpu.async_copy(gidx_hbm.at[pl.ds(grow + b + 3, 1)],
                                     ring_v.at[pl.ds(j, 1)], gsem[j])
                iwait(xsem, j)
                pltpu.async_copy(slots_v.at[j], acc.at[ring_v.at[4 + j]],
                                 ssem[j], add=True)

            def group(i, _):
                for j in range(NSLOT):
                    step(i * NSLOT + j, j)
                return 0
            lax.fori_loop(0, nb // NSLOT, group, 0)
            for j in range(nb - nb // NSLOT * NSLOT):
                step(nb // NSLOT * NSLOT + j, j)
            dwait(ssem, (nb - 1) % NSLOT)
        plsc.subcore_barrier()

        blk = ob + c

        @pl.when(s < N_SUB - 1)
        def _():
            pltpu.sync_copy(acc.at[pl.ds(s * wb, wb)],
                            out_hbm.at[pl.ds(blk * n + s * wb, wb)])

        @pl.when(s == N_SUB - 1)
        def _():
            pltpu.sync_copy(acc.at[pl.ds((N_SUB - 1) * wb, wb_last)],
                            out_hbm.at[pl.ds(blk * n + (N_SUB - 1) * wb, wb_last)])
        plsc.subcore_barrier()


def _fused_call(table, gidx, sidx, zeros, n, nb, passes, n_blocks):
    f = pl.kernel(
        functools.partial(_fused_body, n, nb, passes),
        out_type=jax.ShapeDtypeStruct((n_blocks * n, LANES), jnp.float32),
        mesh=_mesh(),
        scratch_types=[
            pltpu.VMEM((8, 128), jnp.int32),
            pltpu.VMEM((NSLOT, BW, LANES), jnp.float32),
        ] + [pltpu.SemaphoreType.DMA] * 12 + [
            pltpu.VMEM_SHARED((n, LANES), jnp.float32),
        ],
    )
    return f(table, gidx, sidx, zeros)


def _tc_body(nc_in, nc_out, relu, padc, ain, aout, xc, cin, cout, w, b, out):
    rr = ain.shape[2]
    rows = lax.broadcasted_iota(jnp.int32, (rr, 1), 0)
    corr = jnp.where((rows == 0) & (pl.program_id(0) == 0),
                     jnp.float32(padc), jnp.float32(0.0))
    ri = 0.5 / jnp.maximum(cin[0, 0][:, :1] - corr, 1.0)
    ro = 0.5 / jnp.maximum(cout[0, 0][:, :1] - corr, 1.0)
    parts = []
    for k in range(nc_in):
        parts.append((ain[k, 0] - corr * xc[k, 0]) * ri)
    for k in range(nc_in):
        parts.append((aout[k, 0] - corr * xc[k, 0]) * ro)
    for k in range(nc_in):
        parts.append(xc[k, 0])
    cat = jnp.concatenate(parts, axis=1)
    acc = jnp.dot(cat, w[...], preferred_element_type=jnp.float32)
    acc = acc + b[0][None, :]
    if relu:
        acc = jnp.maximum(acc, 0.0)
    if nc_out == 0:
        out[...] = acc
    else:
        for k in range(nc_out):
            out[k, 0] = acc[:, k * 128:(k + 1) * 128]


def _tc_call(aggbuf, xc, cntbuf, w, b, *, nc_in, nc_out, relu, n, grid_r,
             padc, cb):
    R = n // grid_r
    d_out = w.shape[1]
    na = aggbuf.shape[0] // n
    ncb = cntbuf.shape[0] // n
    agg4 = aggbuf.reshape(na, grid_r, R, 128)
    xc4 = xc.reshape(nc_in, grid_r, R, 128)
    cnt4 = cntbuf.reshape(ncb, grid_r, R, 128)
    in_specs = [
        pl.BlockSpec((nc_in, 1, R, 128), lambda i: (0, i, 0, 0)),
        pl.BlockSpec((nc_in, 1, R, 128), lambda i: (1, i, 0, 0)),
        pl.BlockSpec((nc_in, 1, R, 128), lambda i: (0, i, 0, 0)),
        pl.BlockSpec((1, 1, R, 128), lambda i: (cb, i, 0, 0)),
        pl.BlockSpec((1, 1, R, 128), lambda i: (cb + 1, i, 0, 0)),
        pl.BlockSpec(w.shape, lambda i: (0, 0)),
        pl.BlockSpec((1, d_out), lambda i: (0, 0)),
    ]
    if nc_out == 0:
        out_shape = jax.ShapeDtypeStruct((n, d_out), jnp.float32)
        out_spec = pl.BlockSpec((R, d_out), lambda i: (i, 0))
    else:
        out_shape = jax.ShapeDtypeStruct((nc_out * n, 128), jnp.float32)
        out_spec = pl.BlockSpec((nc_out, 1, R, 128), lambda i: (0, i, 0, 0))
    out = pl.pallas_call(
        functools.partial(_tc_body, nc_in, nc_out, relu, padc),
        grid=(grid_r,),
        in_specs=in_specs,
        out_specs=out_spec,
        out_shape=(out_shape if nc_out == 0 else
                   jax.ShapeDtypeStruct((nc_out, grid_r, R, 128), jnp.float32)),
    )(agg4, agg4, xc4, cnt4, cnt4, w, b)
    if nc_out != 0:
        out = out.reshape(nc_out * n, 128)
    return out


def _chunk_major(a, n_chunks):
    n, d = a.shape
    return a.reshape(n, n_chunks, d // n_chunks).transpose(1, 0, 2).reshape(
        n_chunks * n, d // n_chunks)


def _wcat(win, wout, wr):
    return jnp.concatenate([win.T, wout.T, wr.T], axis=0)


def kernel(x, edge_index, Win1, bin1, Wout1, bout1, Wr1, br1,
           Win2, bin2, Wout2, bout2, Wr2, br2):
    n, d_in = x.shape
    e = edge_index.shape[1]
    src = edge_index[0].astype(jnp.int32)
    dst = edge_index[1].astype(jnp.int32)

    ept = e // N_SUB
    nb = (ept + BW - 1) // BW
    pad = nb * BW - ept
    padc = N_SUB * pad

    def tile_batches(idx):
        a = idx.reshape(N_SUB, ept)
        a = jnp.pad(a, ((0, 0), (0, pad)))
        return a.reshape(N_SUB, nb, BW)

    sb_in = tile_batches(dst)
    sb_out = tile_batches(src)
    gb_in = tile_batches(src)
    gb_out = tile_batches(dst)
    sidx = jnp.concatenate([sb_in, sb_out], axis=0).reshape(2 * N_SUB * nb, BW)
    zeros = jnp.zeros((640, LANES), jnp.float32)

    def gidx_for(n_chunks):
        return jnp.concatenate(
            [gb_in + k * n for k in range(n_chunks)]
            + [gb_out + k * n for k in range(n_chunks)], axis=0
        ).reshape(2 * n_chunks * N_SUB * nb, BW)

    grid_r = 10

    c1 = d_in // LANES
    x_t = _chunk_major(x, c1)
    passes1 = (("ones", None, None, 2 * c1),
               ("data", 0, 0, 0), ("data", c1, 1, c1))
    buf1 = _fused_call(x_t, gidx_for(c1), sidx, zeros, n, nb, passes1,
                       2 * c1 + 2)

    d_hid = Win1.shape[0]
    c2 = d_hid // LANES
    w1 = _wcat(Win1, Wout1, Wr1)
    b1 = (0.5 * bin1 + 0.5 * bout1 + br1).reshape(1, -1)
    h_t = _tc_call(buf1, x_t, buf1, w1, b1, nc_in=c1, nc_out=c2, relu=True,
                   n=n, grid_r=grid_r, padc=padc, cb=2 * c1)

    passes2 = tuple(("data", 2 * p, 0, 2 * p) for p in range(c2 // N_CORES)) \
        + tuple(("data", c2 + 2 * p, 1, c2 + 2 * p) for p in range(c2 // N_CORES))
    buf2 = _fused_call(h_t, gidx_for(c2), sidx, zeros, n, nb, passes2, 2 * c2)

    w2 = _wcat(Win2, Wout2, Wr2)
    b2 = (0.5 * bin2 + 0.5 * bout2 + br2).reshape(1, -1)
    out = _tc_call(buf2, h_t, buf1, w2, b2, nc_in=c2, nc_out=0, relu=False,
                   n=n, grid_r=grid_r, padc=padc, cb=2 * c1)
    return out
